# Initial kernel scaffold; baseline (speedup 1.0000x reference)
import jax
import jax.numpy as jnp
from jax.experimental import pallas as pl


def kernel(x, edge_index, W_iou, U_iou, b_iou, W_f, U_f, b_f, W1, b1, W2, b2):
    raise NotImplementedError("write your pallas kernel here")


# speedup vs baseline: 2.0943x; 2.0943x over previous
import functools

import jax
import jax.numpy as jnp
from jax import lax
from jax.experimental import pallas as pl
from jax.experimental.pallas import tpu as pltpu
from jax.experimental.pallas import tpu_sc as plsc


def _sigmoid(v):
    return 1.0 / (1.0 + jnp.exp(-v))


def _dense1_body(x_ref, wiou_ref, biou_ref, wf_ref, bf_ref, uf_ref,
                 xiou_ref, ts_ref, td_ref, h):
    xv = x_ref[...]
    npad = xv.shape[0]
    x_iou = jnp.dot(xv, wiou_ref[...], preferred_element_type=jnp.float32)
    x_iou = x_iou + biou_ref[...]
    x_f = jnp.dot(xv, wf_ref[...], preferred_element_type=jnp.float32)
    x_f = x_f + bf_ref[...]
    xiou_ref[...] = x_iou
    i_g = _sigmoid(x_iou[:, :h])
    o_g = _sigmoid(x_iou[:, h:2 * h])
    u_g = jnp.tanh(x_iou[:, 2 * h:])
    c1 = i_g * u_g
    h1 = o_g * jnp.tanh(c1)
    hu = jnp.dot(h1, uf_ref[...], preferred_element_type=jnp.float32)
    pad = jnp.zeros((npad, h), jnp.float32)
    ts_ref[...] = jnp.concatenate([h1, hu, c1, pad], axis=1)
    td_ref[...] = jnp.concatenate([x_f, pad], axis=1)


def _dense1(xp, W_iou, b_iou, W_f, b_f, U_f):
    npad, d_in = xp.shape
    h = U_f.shape[0]
    return pl.pallas_call(
        functools.partial(_dense1_body, h=h),
        out_shape=(
            jax.ShapeDtypeStruct((npad, 3 * h), jnp.float32),
            jax.ShapeDtypeStruct((npad, 4 * h), jnp.float32),
            jax.ShapeDtypeStruct((npad, 2 * h), jnp.float32),
        ),
    )(xp, W_iou, b_iou.reshape(1, -1), W_f, b_f.reshape(1, -1), U_f)


def _edge_sweep(table, xf, src, dst, *, acc_rows, ch, h):
    e_pad = src.shape[0]
    ept = e_pad // 16
    n_chunks = ept // ch
    half = acc_rows // 2
    local_rows = half + 512
    zrows_per_tile = local_rows // 16
    orows_per_tile = half // 16
    mesh = plsc.VectorSubcoreMesh(core_axis_name="c", subcore_axis_name="s")

    @functools.partial(
        pl.kernel,
        out_type=jax.ShapeDtypeStruct((acc_rows, 2 * h), jnp.float32),
        mesh=mesh,
        scratch_types=[
            pltpu.VMEM((ch,), jnp.int32),
            pltpu.VMEM((ch,), jnp.int32),
            pltpu.VMEM((ch,), jnp.int32),
            pltpu.VMEM((ch, 4 * h), jnp.float32),
            pltpu.VMEM((ch, 2 * h), jnp.float32),
            pltpu.VMEM((ch, 2 * h), jnp.float32),
            pltpu.VMEM_SHARED((local_rows, 2 * h), jnp.float32),
        ],
    )
    def sweep(table_hbm, xf_hbm, src_hbm, dst_hbm, out_hbm,
              src_v, dst_v, tgt_v, rows_v, xf_v, out_v, acc_sh):
        cid = lax.axis_index("c")
        sid = lax.axis_index("s")
        row0 = cid * half
        z16 = jnp.zeros((16,), jnp.float32)

        def zero_row(r, _):
            for j in range(2 * h // 16):
                out_v[r, pl.ds(j * 16, 16)] = z16
            return 0

        lax.fori_loop(0, ch, zero_row, 0)
        zbase = sid * zrows_per_tile
        zleft = zrows_per_tile
        while zleft > 0:
            step = min(zleft, ch)
            pltpu.sync_copy(out_v.at[pl.ds(0, step)],
                            acc_sh.at[pl.ds(zbase + zrows_per_tile - zleft,
                                            step)])
            zleft -= step
        plsc.subcore_barrier()

        def fix_idx(g, _):
            d = dst_v[pl.ds(g * 16, 16)]
            t = d - row0
            oob = (t < 0) | (t >= half)
            tgt_v[pl.ds(g * 16, 16)] = jnp.where(oob, half, t)
            return 0

        def edge_row(r, _):
            for j in range(h // 16):
                xfv = xf_v[r, pl.ds(j * 16, 16)]
                huv = rows_v[r, pl.ds(h + j * 16, 16)]
                cv = rows_v[r, pl.ds(2 * h + j * 16, 16)]
                hv = rows_v[r, pl.ds(j * 16, 16)]
                f = 1.0 / (1.0 + jnp.exp(-(xfv + huv)))
                out_v[r, pl.ds(j * 16, 16)] = hv
                out_v[r, pl.ds(h + j * 16, 16)] = f * cv
            return 0

        ebase = sid * ept

        def chunk(k, _):
            base = pl.multiple_of(ebase + k * ch, 8)
            pltpu.sync_copy(src_hbm.at[pl.ds(base, ch)], src_v)
            pltpu.sync_copy(dst_hbm.at[pl.ds(base, ch)], dst_v)
            pltpu.sync_copy(table_hbm.at[src_v], rows_v)
            pltpu.sync_copy(xf_hbm.at[dst_v], xf_v)
            lax.fori_loop(0, ch // 16, fix_idx, 0)
            lax.fori_loop(0, ch, edge_row, 0)
            pltpu.sync_copy(out_v, acc_sh.at[tgt_v], add=True)
            return 0

        lax.fori_loop(0, n_chunks, chunk, 0)
        plsc.subcore_barrier()
        obase = sid * orows_per_tile
        gbase = pl.multiple_of(row0 + obase, 8)
        pltpu.sync_copy(acc_sh.at[pl.ds(obase, orows_per_tile)],
                        out_hbm.at[pl.ds(gbase, orows_per_tile)])

    return sweep(table, xf, src, dst)


def _dense2_body(xiou_ref, acc_ref, uiou_ref, w1_ref, b1_ref,
                 w2_ref, b2_ref, logits_ref, prob_ref, pred_ref, h):
    acc = acc_ref[...]
    m = acc[:, :h]
    c_sum = acc[:, h:]
    iou = xiou_ref[...] + jnp.dot(m, uiou_ref[...],
                                  preferred_element_type=jnp.float32)
    i_g = _sigmoid(iou[:, :h])
    o_g = _sigmoid(iou[:, h:2 * h])
    u_g = jnp.tanh(iou[:, 2 * h:])
    c2 = i_g * u_g + c_sum
    h2 = o_g * jnp.tanh(c2)
    hidden = jnp.tanh(jnp.dot(h2, w1_ref[...],
                              preferred_element_type=jnp.float32) + b1_ref[...])
    logits = jnp.dot(hidden, w2_ref[...],
                     preferred_element_type=jnp.float32) + b2_ref[...]
    logits_ref[...] = logits
    n_rel = logits.shape[1]
    lmax = jnp.max(logits, axis=1, keepdims=True)
    z = jnp.sum(jnp.exp(logits - lmax), axis=1, keepdims=True)
    prob_ref[...] = 1.0 / z
    col = lax.broadcasted_iota(jnp.int32, logits.shape, 1)
    pred_ref[...] = jnp.min(jnp.where(logits == lmax, col, n_rel),
                            axis=1, keepdims=True)


def _dense2(x_iou, acc, U_iou, W1, b1, W2, b2):
    npad = x_iou.shape[0]
    h = U_iou.shape[0]
    n_rel = W2.shape[1]
    return pl.pallas_call(
        functools.partial(_dense2_body, h=h),
        out_shape=(
            jax.ShapeDtypeStruct((npad, n_rel), jnp.float32),
            jax.ShapeDtypeStruct((npad, 1), jnp.float32),
            jax.ShapeDtypeStruct((npad, 1), jnp.int32),
        ),
    )(x_iou, acc, U_iou, W1, b1.reshape(1, -1), W2, b2.reshape(1, -1))


def kernel(x, edge_index, W_iou, U_iou, b_iou, W_f, U_f, b_f, W1, b1, W2, b2):
    n, _ = x.shape
    h = U_iou.shape[0]
    e = edge_index.shape[1]

    ch = 128
    e_pad = 16 * ch * (-(-e // (16 * ch)))
    n_pad = -(-(n + 1) // 16) * 16
    acc_rows = -(-(n + 1) // (16 * ch)) * 16 * ch

    xp = jnp.pad(x, ((0, n_pad - n), (0, 0)))
    src = jnp.concatenate(
        [edge_index[0], jnp.full((e_pad - e,), n, jnp.int32)])
    dst = jnp.concatenate(
        [edge_index[1], jnp.full((e_pad - e,), n, jnp.int32)])

    x_iou, table_s, table_d = _dense1(xp, W_iou, b_iou, W_f, b_f, U_f)
    acc = _edge_sweep(table_s, table_d, src, dst,
                      acc_rows=acc_rows, ch=ch, h=h)
    logits, prob, pred = _dense2(x_iou, acc[:n_pad],
                                 U_iou, W1, b1, W2, b2)
    return logits[:n], prob[:n, 0], pred[:n, 0]


# speedup vs baseline: 6.2481x; 2.9835x over previous
import functools

import jax
import jax.numpy as jnp
from jax import lax
from jax.experimental import pallas as pl
from jax.experimental.pallas import tpu as pltpu
from jax.experimental.pallas import tpu_sc as plsc


def _sigmoid(v):
    return 1.0 / (1.0 + jnp.exp(-v))


def _dense1_body(x_ref, wiou_ref, biou_ref, wf_ref, bf_ref, uf_ref,
                 xiou_ref, ts_ref, tu_ref, td_ref, h):
    xv = x_ref[...]
    npad = xv.shape[0]
    x_iou = jnp.dot(xv, wiou_ref[...], preferred_element_type=jnp.float32)
    x_iou = x_iou + biou_ref[...]
    x_f = jnp.dot(xv, wf_ref[...], preferred_element_type=jnp.float32)
    x_f = x_f + bf_ref[...]
    xiou_ref[...] = x_iou
    i_g = _sigmoid(x_iou[:, :h])
    o_g = _sigmoid(x_iou[:, h:2 * h])
    u_g = jnp.tanh(x_iou[:, 2 * h:])
    c1 = i_g * u_g
    h1 = o_g * jnp.tanh(c1)
    hu = jnp.dot(h1, uf_ref[...], preferred_element_type=jnp.float32)
    pad = jnp.zeros((npad, h), jnp.float32)
    ts_ref[...] = jnp.concatenate([h1, c1], axis=1)
    tu_ref[...] = jnp.concatenate([hu, pad], axis=1)
    td_ref[...] = jnp.concatenate([x_f, pad], axis=1)


def _dense1(xp, W_iou, b_iou, W_f, b_f, U_f):
    npad, d_in = xp.shape
    h = U_f.shape[0]
    return pl.pallas_call(
        functools.partial(_dense1_body, h=h),
        out_shape=(
            jax.ShapeDtypeStruct((npad, 3 * h), jnp.float32),
            jax.ShapeDtypeStruct((npad, 2 * h), jnp.float32),
            jax.ShapeDtypeStruct((npad, 2 * h), jnp.float32),
            jax.ShapeDtypeStruct((npad, 2 * h), jnp.float32),
        ),
    )(xp, W_iou, b_iou.reshape(1, -1), W_f, b_f.reshape(1, -1), U_f)


def _edge_sweep(t_hc, t_hu, t_d, src, dst, *, acc_rows, ch, h):
    e_pad = src.shape[0]
    ept = e_pad // 16
    n_chunks = ept // ch
    n_pairs = n_chunks // 2
    half = acc_rows // 2
    local_rows = half + 128
    zrows_per_tile = local_rows // 16
    orows_per_tile = half // 16
    mesh = plsc.VectorSubcoreMesh(core_axis_name="c", subcore_axis_name="s")

    @functools.partial(
        pl.kernel,
        out_type=jax.ShapeDtypeStruct((acc_rows, 2 * h), jnp.float32),
        mesh=mesh,
        scratch_types=[
            pltpu.VMEM((ch,), jnp.int32),
            pltpu.VMEM((ch,), jnp.int32),
            pltpu.VMEM((ch,), jnp.int32),
            pltpu.VMEM((ch,), jnp.int32),
            pltpu.VMEM((ch,), jnp.int32),
            pltpu.VMEM((ch,), jnp.int32),
            pltpu.VMEM((ch, 2 * h), jnp.float32),
            pltpu.VMEM((ch, 2 * h), jnp.float32),
            pltpu.VMEM((ch, 2 * h), jnp.float32),
            pltpu.VMEM((ch, 2 * h), jnp.float32),
            pltpu.VMEM((ch, 2 * h), jnp.float32),
            pltpu.VMEM((ch, 2 * h), jnp.float32),
            pltpu.VMEM_SHARED((local_rows, 2 * h), jnp.float32),
            pltpu.SemaphoreType.DMA,
            pltpu.SemaphoreType.DMA,
            pltpu.SemaphoreType.DMA,
            pltpu.SemaphoreType.DMA,
            pltpu.SemaphoreType.DMA,
            pltpu.SemaphoreType.DMA,
        ],
    )
    def sweep(hc_hbm, hu_hbm, xf_hbm, src_hbm, dst_hbm, out_hbm,
              src0, src1, dst0, dst1, tgt0, tgt1,
              hc0, hc1, hu0, hu1, xf0, xf1, acc_sh,
              si0, si1, sg0, sg1, ss0, ss1):
        cid = lax.axis_index("c")
        sid = lax.axis_index("s")
        row0 = cid * half
        ebase = sid * ept
        z16 = jnp.zeros((16,), jnp.float32)
        srcb, dstb, tgtb = (src0, src1), (dst0, dst1), (tgt0, tgt1)
        hcb, hub, xfb = (hc0, hc1), (hu0, hu1), (xf0, xf1)
        sib, sgb, ssb = (si0, si1), (sg0, sg1), (ss0, ss1)

        def zero_row(r, _):
            for j in range(2 * h // 16):
                hc0[r, pl.ds(j * 16, 16)] = z16
            return 0

        lax.fori_loop(0, ch, zero_row, 0)
        zbase = sid * zrows_per_tile
        zleft = zrows_per_tile
        while zleft > 0:
            step = min(zleft, ch)
            pltpu.sync_copy(hc0.at[pl.ds(0, step)],
                            acc_sh.at[pl.ds(zbase + zrows_per_tile - zleft,
                                            step)])
            zleft -= step

        def idx_start(k, p):
            base = pl.multiple_of(ebase + k * ch, 8)
            pltpu.async_copy(src_hbm.at[pl.ds(base, ch)], srcb[p], sib[p])
            pltpu.async_copy(dst_hbm.at[pl.ds(base, ch)], dstb[p], sib[p])

        def idx_wait(k, p):
            base = pl.multiple_of(ebase + k * ch, 8)
            pltpu.make_async_copy(src_hbm.at[pl.ds(base, ch)], srcb[p],
                                  sib[p]).wait()
            pltpu.make_async_copy(dst_hbm.at[pl.ds(base, ch)], dstb[p],
                                  sib[p]).wait()

        def gathers_start(p):
            pltpu.async_copy(hc_hbm.at[srcb[p]], hcb[p], sgb[p])
            pltpu.async_copy(hu_hbm.at[srcb[p]], hub[p], sgb[p])
            pltpu.async_copy(xf_hbm.at[dstb[p]], xfb[p], sgb[p])

        def gathers_wait(p):
            pltpu.make_async_copy(hc_hbm.at[srcb[p]], hcb[p], sgb[p]).wait()
            pltpu.make_async_copy(hu_hbm.at[srcb[p]], hub[p], sgb[p]).wait()
            pltpu.make_async_copy(xf_hbm.at[dstb[p]], xfb[p], sgb[p]).wait()

        def scatter_start(p):
            pltpu.async_copy(hcb[p], acc_sh.at[tgtb[p]], ssb[p], add=True)

        def scatter_wait(p):
            pltpu.make_async_copy(hcb[p], acc_sh.at[tgtb[p]], ssb[p]).wait()

        def make_fix_idx(p):
            def fix_idx(g, _):
                d = dstb[p][pl.ds(g * 16, 16)]
                t = d - row0
                oob = (t < 0) | (t >= half)
                tgtb[p][pl.ds(g * 16, 16)] = jnp.where(oob, half, t)
                return 0
            return fix_idx

        def make_edge_row(p):
            hc_v, hu_v, xf_v = hcb[p], hub[p], xfb[p]

            def edge_row(r, _):
                for j in range(h // 16):
                    xfv = xf_v[r, pl.ds(j * 16, 16)]
                    huv = hu_v[r, pl.ds(j * 16, 16)]
                    cv = hc_v[r, pl.ds(h + j * 16, 16)]
                    hc_v[r, pl.ds(h + j * 16, 16)] = (
                        cv / (1.0 + jnp.exp(-(xfv + huv))))
                return 0
            return edge_row

        idx_start(0, 0)
        idx_wait(0, 0)
        gathers_start(0)
        plsc.subcore_barrier()

        def chunk_step(k, p):
            @pl.when(k + 1 < n_chunks)
            def _():
                idx_start(k + 1, 1 - p)

            gathers_wait(p)
            lax.fori_loop(0, ch // 16, make_fix_idx(p), 0)
            lax.fori_loop(0, ch, make_edge_row(p), 0)

            @pl.when(k > 0)
            def _():
                scatter_wait(1 - p)

            @pl.when(k + 1 < n_chunks)
            def _():
                idx_wait(k + 1, 1 - p)
                gathers_start(1 - p)

            scatter_start(p)

        def pair(i, _):
            chunk_step(2 * i, 0)
            chunk_step(2 * i + 1, 1)
            return 0

        lax.fori_loop(0, n_pairs, pair, 0)
        scatter_wait(1)
        plsc.subcore_barrier()
        obase = sid * orows_per_tile
        gbase = pl.multiple_of(row0 + obase, 8)
        pltpu.sync_copy(acc_sh.at[pl.ds(obase, orows_per_tile)],
                        out_hbm.at[pl.ds(gbase, orows_per_tile)])

    return sweep(t_hc, t_hu, t_d, src, dst)


def _dense2_body(xiou_ref, acc_ref, uiou_ref, w1_ref, b1_ref,
                 w2_ref, b2_ref, logits_ref, prob_ref, pred_ref, h):
    acc = acc_ref[...]
    m = acc[:, :h]
    c_sum = acc[:, h:]
    iou = xiou_ref[...] + jnp.dot(m, uiou_ref[...],
                                  preferred_element_type=jnp.float32)
    i_g = _sigmoid(iou[:, :h])
    o_g = _sigmoid(iou[:, h:2 * h])
    u_g = jnp.tanh(iou[:, 2 * h:])
    c2 = i_g * u_g + c_sum
    h2 = o_g * jnp.tanh(c2)
    hidden = jnp.tanh(jnp.dot(h2, w1_ref[...],
                              preferred_element_type=jnp.float32) + b1_ref[...])
    logits = jnp.dot(hidden, w2_ref[...],
                     preferred_element_type=jnp.float32) + b2_ref[...]
    logits_ref[...] = logits
    n_rel = logits.shape[1]
    lmax = jnp.max(logits, axis=1, keepdims=True)
    z = jnp.sum(jnp.exp(logits - lmax), axis=1, keepdims=True)
    prob_ref[...] = 1.0 / z
    col = lax.broadcasted_iota(jnp.int32, logits.shape, 1)
    pred_ref[...] = jnp.min(jnp.where(logits == lmax, col, n_rel),
                            axis=1, keepdims=True)


def _dense2(x_iou, acc, U_iou, W1, b1, W2, b2):
    npad = x_iou.shape[0]
    h = U_iou.shape[0]
    n_rel = W2.shape[1]
    return pl.pallas_call(
        functools.partial(_dense2_body, h=h),
        out_shape=(
            jax.ShapeDtypeStruct((npad, n_rel), jnp.float32),
            jax.ShapeDtypeStruct((npad, 1), jnp.float32),
            jax.ShapeDtypeStruct((npad, 1), jnp.int32),
        ),
    )(x_iou, acc, U_iou, W1, b1.reshape(1, -1), W2, b2.reshape(1, -1))


def kernel(x, edge_index, W_iou, U_iou, b_iou, W_f, U_f, b_f, W1, b1, W2, b2):
    n, _ = x.shape
    h = U_iou.shape[0]
    e = edge_index.shape[1]

    ch = 112
    e_pad = 32 * ch * (-(-e // (32 * ch)))
    n_pad = -(-(n + 1) // 16) * 16
    acc_rows = -(-(n + 1) // 2048) * 2048

    xp = jnp.pad(x, ((0, n_pad - n), (0, 0)))
    src = jnp.concatenate(
        [edge_index[0], jnp.full((e_pad - e,), n, jnp.int32)])
    dst = jnp.concatenate(
        [edge_index[1], jnp.full((e_pad - e,), n, jnp.int32)])

    x_iou, t_hc, t_hu, t_d = _dense1(xp, W_iou, b_iou, W_f, b_f, U_f)
    acc = _edge_sweep(t_hc, t_hu, t_d, src, dst,
                      acc_rows=acc_rows, ch=ch, h=h)
    logits, prob, pred = _dense2(x_iou, acc[:n_pad],
                                 U_iou, W1, b1, W2, b2)
    return logits[:n], prob[:n, 0], pred[:n, 0]
